# baseline (speedup 1.0000x reference)
import jax
import jax.numpy as jnp
from jax.experimental import pallas as pl
from jax.experimental.pallas import tpu as pltpu

_ET = 64
_CB = 2048


def _leaky(t):
    return jnp.where(t >= 0.0, t, 0.01 * t)


def _hwd_body(ea_ref, w1_ref, b1_ref, w2_ref, b2_ref, o_ref, h_scr):
    @pl.when(pl.program_id(0) == 0)
    def _():
        t = jax.lax.dot_general(
            ea_ref[...], w1_ref[...], (((1,), (0,)), ((), ())),
            precision=jax.lax.Precision.DEFAULT,
            preferred_element_type=jnp.float32)
        h_scr[...] = _leaky(t + b1_ref[...][None, :]).astype(jnp.bfloat16)

    t2 = jax.lax.dot_general(
        h_scr[...], w2_ref[...], (((1,), (0,)), ((), ())),
        precision=jax.lax.Precision.DEFAULT,
        preferred_element_type=jnp.float32)
    o_ref[...] = (t2 + b2_ref[...]).astype(jnp.bfloat16)


def _pallas_wd(ea, W1, b1, W2, b2):
    e, fin = ea.shape
    hid = W1.shape[1]
    cols = W2.shape[1]
    b2r = b2.reshape(1, cols)
    grid = cols // _CB
    return pl.pallas_call(
        _hwd_body,
        grid=(grid,),
        in_specs=[
            pl.BlockSpec((e, fin), lambda c: (0, 0)),
            pl.BlockSpec((fin, hid), lambda c: (0, 0)),
            pl.BlockSpec((hid,), lambda c: (0,)),
            pl.BlockSpec((hid, _CB), lambda c: (0, c)),
            pl.BlockSpec((1, _CB), lambda c: (0, c)),
        ],
        out_specs=pl.BlockSpec((e, _CB), lambda c: (0, c)),
        out_shape=jax.ShapeDtypeStruct((e, cols), jnp.bfloat16),
        scratch_shapes=[pltpu.VMEM((e, hid), jnp.bfloat16)],
    )(ea, W1, b1, W2, b2r)


def _matvec_body(xj_ref, wd_ref, o_ref):
    for e in range(_ET):
        a = xj_ref[e:e + 1, :]
        w = wd_ref[e]
        o_ref[e:e + 1, :] = jax.lax.dot_general(
            a, w, (((1,), (0,)), ((), ())),
            precision=jax.lax.Precision.DEFAULT,
            preferred_element_type=jnp.float32)


def _pallas_batched_matvec(xj, Wd16):
    e, in_ch = xj.shape
    out_ch = Wd16.shape[2]
    grid = e // _ET
    return pl.pallas_call(
        _matvec_body,
        grid=(grid,),
        in_specs=[
            pl.BlockSpec((_ET, in_ch), lambda i: (i, 0)),
            pl.BlockSpec((_ET, in_ch, out_ch), lambda i: (i, 0, 0)),
        ],
        out_specs=pl.BlockSpec((_ET, out_ch), lambda i: (i, 0)),
        out_shape=jax.ShapeDtypeStruct((e, out_ch), jnp.float32),
    )(xj, Wd16)


def _count_body(dst_ref, o_ref):
    n = o_ref.shape[0]
    nid = jax.lax.broadcasted_iota(jnp.int32, (n, 1), 0)
    eq = (dst_ref[0:1, :] == nid).astype(jnp.float32)
    o_ref[...] = jnp.sum(eq, axis=1, keepdims=True)


def _pallas_counts(dst, n):
    e = dst.shape[0]
    dst2d = jnp.broadcast_to(dst[None, :], (8, e))
    return pl.pallas_call(
        _count_body,
        out_shape=jax.ShapeDtypeStruct((n, 1), jnp.float32),
    )(dst2d)


def _update_body_factory(to_bf16):
    def body(s_ref, c_ref, xr_ref, root_ref, bias_ref, o_ref):
        mean = s_ref[...] / jnp.maximum(c_ref[...], 1.0)
        rootterm = jax.lax.dot_general(
            xr_ref[...], root_ref[...], (((1,), (0,)), ((), ())),
            precision=jax.lax.Precision.DEFAULT,
            preferred_element_type=jnp.float32)
        t = (mean + rootterm) + bias_ref[...][None, :]
        d = _leaky(t)
        o_ref[...] = d.astype(jnp.bfloat16) if to_bf16 else d
    return body


def _pallas_update(s, c, xr, root, bias, to_bf16):
    n, out_ch = s.shape
    dt = jnp.bfloat16 if to_bf16 else jnp.float32
    return pl.pallas_call(
        _update_body_factory(to_bf16),
        out_shape=jax.ShapeDtypeStruct((n, out_ch), dt),
    )(s, c, xr, root, bias)


def _final_body(d_ref, g_ref, tau_ref, o_ref):
    logits = (d_ref[...] + g_ref[...]) / tau_ref[...]
    m = jnp.max(logits, axis=1, keepdims=True)
    ex = jnp.exp(logits - m)
    ssum = jnp.sum(ex, axis=1, keepdims=True)
    ys = ex / ssum
    mm = jnp.max(ys, axis=1, keepdims=True)
    n, k = ys.shape
    iota = jax.lax.broadcasted_iota(jnp.int32, (n, k), 1)
    idx = jnp.min(jnp.where(ys == mm, iota, k), axis=1, keepdims=True)
    yh = (iota == idx).astype(jnp.float32)
    o_ref[...] = (yh - ys) + ys


def _pallas_final(d3, g, tau):
    n, k = d3.shape
    return pl.pallas_call(
        _final_body,
        out_shape=jax.ShapeDtypeStruct((n, k), jnp.float32),
    )(d3, g, jnp.broadcast_to(tau, (1, 1)))


def _layer(x_f32, x_b16, src, dst, c, Wd_parts, root, bias, to_bf16):
    n = x_f32.shape[0] if x_f32 is not None else x_b16.shape[0]
    if x_b16 is None:
        xj = jnp.take(x_f32, src, axis=0)
        xr = x_f32
    else:
        xj = jnp.take(x_b16, src, axis=0)
        xr = x_b16
    nparts = len(Wd_parts)
    epp = xj.shape[0] // nparts
    msgs = [_pallas_batched_matvec(xj[i * epp:(i + 1) * epp], Wd_parts[i])
            for i in range(nparts)]
    msg = msgs[0] if nparts == 1 else jnp.concatenate(msgs, axis=0)
    s = jax.ops.segment_sum(msg, dst, num_segments=n)
    return _pallas_update(s, c, xr, root, bias, to_bf16)


def kernel(x, edge_index, edge_attr, epoch,
           nn1_W1, nn1_b1, nn1_W2, nn1_b2, root1, bias1,
           nn2_W1, nn2_b1, nn2_W2, nn2_b2, root2, bias2,
           nn3_W1, nn3_b1, nn3_W2, nn3_b2, root3, bias3):
    src = edge_index[0]
    dst = edge_index[1]
    tau = 500.0 / (epoch + 1)
    n = x.shape[0]
    e = edge_attr.shape[0]
    eh = e // 2
    Wd1 = _pallas_wd(edge_attr, nn1_W1, nn1_b1, nn1_W2, nn1_b2).reshape(e, 64, 512)
    Wd2a = _pallas_wd(edge_attr[:eh], nn2_W1, nn2_b1, nn2_W2, nn2_b2).reshape(eh, 512, 256)
    Wd2b = _pallas_wd(edge_attr[eh:], nn2_W1, nn2_b1, nn2_W2, nn2_b2).reshape(eh, 512, 256)
    Wd3 = _pallas_wd(edge_attr, nn3_W1, nn3_b1, nn3_W2, nn3_b2).reshape(e, 256, 64)
    c = _pallas_counts(dst, n)
    d1_16 = _layer(x, None, src, dst, c, [Wd1], root1, bias1, True)
    d2_16 = _layer(None, d1_16, src, dst, c, [Wd2a, Wd2b], root2, bias2, True)
    d3 = _layer(None, d2_16, src, dst, c, [Wd3], root3, bias3, False)
    g = jax.random.gumbel(jax.random.key(42), d3.shape, dtype=d3.dtype)
    return _pallas_final(d3, g, tau)


# speedup vs baseline: 1.0203x; 1.0203x over previous
import jax
import jax.numpy as jnp
from jax.experimental import pallas as pl
from jax.experimental.pallas import tpu as pltpu

_ET = 64
_CB = 2048


def _leaky(t):
    return jnp.where(t >= 0.0, t, 0.01 * t)


def _hwd_body(ea_ref, w1_ref, b1_ref, w2_ref, b2_ref, o_ref, h_scr):
    @pl.when(pl.program_id(0) == 0)
    def _():
        t = jax.lax.dot_general(
            ea_ref[...], w1_ref[...], (((1,), (0,)), ((), ())),
            precision=jax.lax.Precision.DEFAULT,
            preferred_element_type=jnp.float32)
        h_scr[...] = _leaky(t + b1_ref[...][None, :]).astype(jnp.bfloat16)

    t2 = jax.lax.dot_general(
        h_scr[...], w2_ref[...], (((1,), (0,)), ((), ())),
        precision=jax.lax.Precision.DEFAULT,
        preferred_element_type=jnp.float32)
    o_ref[...] = (t2 + b2_ref[...]).astype(jnp.bfloat16)


def _pallas_wd(ea, W1, b1, W2, b2):
    e, fin = ea.shape
    hid = W1.shape[1]
    cols = W2.shape[1]
    b2r = b2.reshape(1, cols)
    grid = cols // _CB
    return pl.pallas_call(
        _hwd_body,
        grid=(grid,),
        in_specs=[
            pl.BlockSpec((e, fin), lambda c: (0, 0)),
            pl.BlockSpec((fin, hid), lambda c: (0, 0)),
            pl.BlockSpec((hid,), lambda c: (0,)),
            pl.BlockSpec((hid, _CB), lambda c: (0, c)),
            pl.BlockSpec((1, _CB), lambda c: (0, c)),
        ],
        out_specs=pl.BlockSpec((e, _CB), lambda c: (0, c)),
        out_shape=jax.ShapeDtypeStruct((e, cols), jnp.bfloat16),
        scratch_shapes=[pltpu.VMEM((e, hid), jnp.bfloat16)],
    )(ea, W1, b1, W2, b2r)


def _matvec_body(xj_ref, wd_ref, o_ref):
    for e in range(_ET):
        a = xj_ref[e:e + 1, :]
        w = wd_ref[e]
        o_ref[e:e + 1, :] = jax.lax.dot_general(
            a, w, (((1,), (0,)), ((), ())),
            precision=jax.lax.Precision.DEFAULT,
            preferred_element_type=jnp.float32)


def _pallas_batched_matvec(xj, Wd16):
    e, in_ch = xj.shape
    out_ch = Wd16.shape[2]
    grid = e // _ET
    return pl.pallas_call(
        _matvec_body,
        grid=(grid,),
        in_specs=[
            pl.BlockSpec((_ET, in_ch), lambda i: (i, 0)),
            pl.BlockSpec((_ET, in_ch, out_ch), lambda i: (i, 0, 0)),
        ],
        out_specs=pl.BlockSpec((_ET, out_ch), lambda i: (i, 0)),
        out_shape=jax.ShapeDtypeStruct((e, out_ch), jnp.float32),
    )(xj, Wd16)


def _count_body(dst_ref, o_ref):
    n = o_ref.shape[0]
    nid = jax.lax.broadcasted_iota(jnp.int32, (n, 1), 0)
    eq = (dst_ref[0:1, :] == nid).astype(jnp.float32)
    o_ref[...] = jnp.sum(eq, axis=1, keepdims=True)


def _pallas_counts(dst, n):
    e = dst.shape[0]
    dst2d = jnp.broadcast_to(dst[None, :], (8, e))
    return pl.pallas_call(
        _count_body,
        out_shape=jax.ShapeDtypeStruct((n, 1), jnp.float32),
    )(dst2d)


def _update_body_factory(to_bf16):
    def body(s_ref, c_ref, xr_ref, root_ref, bias_ref, o_ref):
        mean = s_ref[...] / jnp.maximum(c_ref[...], 1.0)
        rootterm = jax.lax.dot_general(
            xr_ref[...], root_ref[...], (((1,), (0,)), ((), ())),
            precision=jax.lax.Precision.DEFAULT,
            preferred_element_type=jnp.float32)
        t = (mean + rootterm) + bias_ref[...][None, :]
        d = _leaky(t)
        o_ref[...] = d.astype(jnp.bfloat16) if to_bf16 else d
    return body


def _pallas_update(s, c, xr, root, bias, to_bf16):
    n, out_ch = s.shape
    dt = jnp.bfloat16 if to_bf16 else jnp.float32
    return pl.pallas_call(
        _update_body_factory(to_bf16),
        out_shape=jax.ShapeDtypeStruct((n, out_ch), dt),
    )(s, c, xr, root, bias)


def _final_body(d_ref, g_ref, tau_ref, o_ref):
    logits = (d_ref[...] + g_ref[...]) / tau_ref[...]
    m = jnp.max(logits, axis=1, keepdims=True)
    ex = jnp.exp(logits - m)
    ssum = jnp.sum(ex, axis=1, keepdims=True)
    ys = ex / ssum
    mm = jnp.max(ys, axis=1, keepdims=True)
    n, k = ys.shape
    iota = jax.lax.broadcasted_iota(jnp.int32, (n, k), 1)
    idx = jnp.min(jnp.where(ys == mm, iota, k), axis=1, keepdims=True)
    yh = (iota == idx).astype(jnp.float32)
    o_ref[...] = (yh - ys) + ys


def _pallas_final(d3, g, tau):
    n, k = d3.shape
    return pl.pallas_call(
        _final_body,
        out_shape=jax.ShapeDtypeStruct((n, k), jnp.float32),
    )(d3, g, jnp.broadcast_to(tau, (1, 1)))


def _layer(x_f32, x_b16, src, dst, c, Wd_parts, root, bias, to_bf16):
    n = x_f32.shape[0] if x_f32 is not None else x_b16.shape[0]
    if x_b16 is None:
        xj = jnp.take(x_f32, src, axis=0)
        xr = x_f32
    else:
        xj = jnp.take(x_b16, src, axis=0)
        xr = x_b16
    nparts = len(Wd_parts)
    epp = xj.shape[0] // nparts
    msgs = [_pallas_batched_matvec(xj[i * epp:(i + 1) * epp], Wd_parts[i])
            for i in range(nparts)]
    msg = msgs[0] if nparts == 1 else jnp.concatenate(msgs, axis=0)
    s = jax.ops.segment_sum(msg, dst, num_segments=n)
    return _pallas_update(s, c, xr, root, bias, to_bf16)


def kernel(x, edge_index, edge_attr, epoch,
           nn1_W1, nn1_b1, nn1_W2, nn1_b2, root1, bias1,
           nn2_W1, nn2_b1, nn2_W2, nn2_b2, root2, bias2,
           nn3_W1, nn3_b1, nn3_W2, nn3_b2, root3, bias3):
    src = edge_index[0]
    dst = edge_index[1]
    tau = 500.0 / (epoch + 1)
    n = x.shape[0]
    e = edge_attr.shape[0]
    eh = e // 2
    Wd1 = _pallas_wd(edge_attr, nn1_W1, nn1_b1, nn1_W2, nn1_b2).reshape(e, 64, 512)
    Wd2 = _pallas_wd(edge_attr, nn2_W1, nn2_b1, nn2_W2, nn2_b2).reshape(e, 512, 256)
    Wd3 = _pallas_wd(edge_attr, nn3_W1, nn3_b1, nn3_W2, nn3_b2).reshape(e, 256, 64)
    c = _pallas_counts(dst, n)
    d1_16 = _layer(x, None, src, dst, c, [Wd1], root1, bias1, True)
    d2_16 = _layer(None, d1_16, src, dst, c, [Wd2], root2, bias2, True)
    d3 = _layer(None, d2_16, src, dst, c, [Wd3], root3, bias3, False)
    g = jax.random.gumbel(jax.random.key(42), d3.shape, dtype=d3.dtype)
    return _pallas_final(d3, g, tau)


# speedup vs baseline: 1.0204x; 1.0000x over previous
import jax
import jax.numpy as jnp
from jax.experimental import pallas as pl
from jax.experimental.pallas import tpu as pltpu

_ET = 64
_CB = 2048


def _leaky(t):
    return jnp.where(t >= 0.0, t, 0.01 * t)


def _hwd_body(ea_ref, w1_ref, b1_ref, w2_ref, b2_ref, o_ref, h_scr):
    @pl.when(pl.program_id(0) == 0)
    def _():
        t = jax.lax.dot_general(
            ea_ref[...], w1_ref[...], (((1,), (0,)), ((), ())),
            precision=jax.lax.Precision.DEFAULT,
            preferred_element_type=jnp.float32)
        h_scr[...] = _leaky(t + b1_ref[...][None, :]).astype(jnp.bfloat16)

    t2 = jax.lax.dot_general(
        h_scr[...], w2_ref[...], (((1,), (0,)), ((), ())),
        precision=jax.lax.Precision.DEFAULT,
        preferred_element_type=jnp.float32)
    o_ref[...] = (t2 + b2_ref[...]).astype(jnp.bfloat16)


def _pallas_wd(ea, W1, b1, W2, b2):
    e, fin = ea.shape
    hid = W1.shape[1]
    cols = W2.shape[1]
    b2r = b2.reshape(1, cols)
    grid = cols // _CB
    return pl.pallas_call(
        _hwd_body,
        grid=(grid,),
        in_specs=[
            pl.BlockSpec((e, fin), lambda c: (0, 0)),
            pl.BlockSpec((fin, hid), lambda c: (0, 0)),
            pl.BlockSpec((hid,), lambda c: (0,)),
            pl.BlockSpec((hid, _CB), lambda c: (0, c)),
            pl.BlockSpec((1, _CB), lambda c: (0, c)),
        ],
        out_specs=pl.BlockSpec((e, _CB), lambda c: (0, c)),
        out_shape=jax.ShapeDtypeStruct((e, cols), jnp.bfloat16),
        scratch_shapes=[pltpu.VMEM((e, hid), jnp.bfloat16)],
    )(ea, W1, b1, W2, b2r)


def _matvec_body(xj_ref, wd_ref, o_ref):
    for e in range(_ET):
        a = xj_ref[e:e + 1, :]
        w = wd_ref[e]
        o_ref[e:e + 1, :] = jax.lax.dot_general(
            a, w, (((1,), (0,)), ((), ())),
            precision=jax.lax.Precision.DEFAULT,
            preferred_element_type=jnp.float32)


def _pallas_batched_matvec(xj, Wd16):
    e, in_ch = xj.shape
    out_ch = Wd16.shape[2]
    grid = e // _ET
    return pl.pallas_call(
        _matvec_body,
        grid=(grid,),
        in_specs=[
            pl.BlockSpec((_ET, in_ch), lambda i: (i, 0)),
            pl.BlockSpec((_ET, in_ch, out_ch), lambda i: (i, 0, 0)),
        ],
        out_specs=pl.BlockSpec((_ET, out_ch), lambda i: (i, 0)),
        out_shape=jax.ShapeDtypeStruct((e, out_ch), jnp.float32),
    )(xj, Wd16)


def _count_body(dst_ref, o_ref):
    n = o_ref.shape[0]
    nid = jax.lax.broadcasted_iota(jnp.int32, (n, 1), 0)
    eq = (dst_ref[0:1, :] == nid).astype(jnp.float32)
    o_ref[...] = jnp.sum(eq, axis=1, keepdims=True)


def _pallas_counts(dst, n):
    e = dst.shape[0]
    dst2d = jnp.broadcast_to(dst[None, :], (8, e))
    return pl.pallas_call(
        _count_body,
        out_shape=jax.ShapeDtypeStruct((n, 1), jnp.float32),
    )(dst2d)


def _update_body_factory(to_bf16):
    def body(s_ref, c_ref, xr_ref, root_ref, bias_ref, o_ref):
        mean = s_ref[...] / jnp.maximum(c_ref[...], 1.0)
        rootterm = jax.lax.dot_general(
            xr_ref[...], root_ref[...], (((1,), (0,)), ((), ())),
            precision=jax.lax.Precision.DEFAULT,
            preferred_element_type=jnp.float32)
        t = (mean + rootterm) + bias_ref[...][None, :]
        d = _leaky(t)
        o_ref[...] = d.astype(jnp.bfloat16) if to_bf16 else d
    return body


def _pallas_update(s, c, xr, root, bias, to_bf16):
    n, out_ch = s.shape
    dt = jnp.bfloat16 if to_bf16 else jnp.float32
    return pl.pallas_call(
        _update_body_factory(to_bf16),
        out_shape=jax.ShapeDtypeStruct((n, out_ch), dt),
    )(s, c, xr, root, bias)


def _final_body(d_ref, g_ref, tau_ref, o_ref):
    logits = (d_ref[...] + g_ref[...]) / tau_ref[...]
    m = jnp.max(logits, axis=1, keepdims=True)
    ex = jnp.exp(logits - m)
    ssum = jnp.sum(ex, axis=1, keepdims=True)
    ys = ex / ssum
    mm = jnp.max(ys, axis=1, keepdims=True)
    n, k = ys.shape
    iota = jax.lax.broadcasted_iota(jnp.int32, (n, k), 1)
    idx = jnp.min(jnp.where(ys == mm, iota, k), axis=1, keepdims=True)
    yh = (iota == idx).astype(jnp.float32)
    o_ref[...] = (yh - ys) + ys


def _pallas_final(d3, g, tau):
    n, k = d3.shape
    return pl.pallas_call(
        _final_body,
        out_shape=jax.ShapeDtypeStruct((n, k), jnp.float32),
    )(d3, g, jnp.broadcast_to(tau, (1, 1)))


def _layer(x_f32, x_b16, src, dst, c, Wd_parts, root, bias, to_bf16):
    n = x_f32.shape[0] if x_f32 is not None else x_b16.shape[0]
    if x_b16 is None:
        xj = jnp.take(x_f32, src, axis=0)
        xr = x_f32
    else:
        xj = jnp.take(x_b16, src, axis=0)
        xr = x_b16
    nparts = len(Wd_parts)
    epp = xj.shape[0] // nparts
    msgs = [_pallas_batched_matvec(xj[i * epp:(i + 1) * epp], Wd_parts[i])
            for i in range(nparts)]
    msg = msgs[0] if nparts == 1 else jnp.concatenate(msgs, axis=0)
    s = jax.ops.segment_sum(msg, dst, num_segments=n)
    return _pallas_update(s, c, xr, root, bias, to_bf16)


def kernel(x, edge_index, edge_attr, epoch,
           nn1_W1, nn1_b1, nn1_W2, nn1_b2, root1, bias1,
           nn2_W1, nn2_b1, nn2_W2, nn2_b2, root2, bias2,
           nn3_W1, nn3_b1, nn3_W2, nn3_b2, root3, bias3):
    src = edge_index[0]
    dst = edge_index[1]
    tau = 500.0 / (epoch + 1)
    n = x.shape[0]
    e = edge_attr.shape[0]
    Wd1 = _pallas_wd(edge_attr, nn1_W1, nn1_b1, nn1_W2, nn1_b2).reshape(e, 64, 512)
    Wd2 = _pallas_wd(edge_attr, nn2_W1, nn2_b1, nn2_W2, nn2_b2).reshape(e, 512, 256)
    Wd3 = _pallas_wd(edge_attr, nn3_W1, nn3_b1, nn3_W2, nn3_b2).reshape(e, 256, 64)
    c = _pallas_counts(dst, n)
    d1_16 = _layer(x, None, src, dst, c, [Wd1], root1, bias1, True)
    d2_16 = _layer(None, d1_16, src, dst, c, [Wd2], root2, bias2, True)
    d3 = _layer(None, d2_16, src, dst, c, [Wd3], root3, bias3, False)
    g = jax.random.gumbel(jax.random.key(42), d3.shape, dtype=d3.dtype)
    return _pallas_final(d3, g, tau)
